# Initial kernel scaffold; baseline (speedup 1.0000x reference)
import jax
import jax.numpy as jnp
from jax.experimental import pallas as pl


def kernel(pred_t, t, step_size, x_t):
    raise NotImplementedError("write your pallas kernel here")


# speedup vs baseline: 1.5655x; 1.5655x over previous
import numpy as np
import jax
import jax.numpy as jnp
from jax.experimental import pallas as pl
from jax.experimental.pallas import tpu as pltpu

N = 262144
D = 119
ROWS = 1024

K_SAMP = (1832780943, 270669613)
K_JUMP = (64467757, 2916123636)
TINY = float(np.finfo(np.float32).tiny)
MAGIC = 0x1BD11BDA

_ROT_A = (13, 15, 26, 6)
_ROT_B = (17, 29, 16, 24)


def _threefry_xor(key, ctr):
    k0, k1 = np.uint32(key[0]), np.uint32(key[1])
    k2 = np.uint32(int(k0) ^ int(k1) ^ MAGIC)
    ks = (jnp.uint32(k0), jnp.uint32(k1), jnp.uint32(k2))
    x0 = jnp.full(ctr.shape, ks[0], jnp.uint32)
    x1 = ctr + ks[1]
    for i in range(5):
        for r in (_ROT_A if i % 2 == 0 else _ROT_B):
            x0 = x0 + x1
            x1 = ((x1 << jnp.uint32(r)) | (x1 >> jnp.uint32(32 - r))) ^ x0
        x0 = x0 + ks[(i + 1) % 3]
        x1 = x1 + ks[(i + 2) % 3] + jnp.uint32(i + 1)
    return x0 ^ x1


def _bits_to_unit_float(bits):
    fb = (bits >> jnp.uint32(9)) | jnp.uint32(0x3F800000)
    return jax.lax.bitcast_convert_type(fb, jnp.float32) - jnp.float32(1.0)


def _body(pred_ref, xt_ref, t_ref, h_ref, out_ref):
    pid = pl.program_id(0)
    r0 = pid * ROWS

    row = jax.lax.broadcasted_iota(jnp.int32, (ROWS, D), 0) + r0
    col = jax.lax.broadcasted_iota(jnp.int32, (ROWS, D), 1)
    ctr = (row * D + col).astype(jnp.uint32)
    bits = _threefry_xor(K_SAMP, ctr)
    f = _bits_to_unit_float(bits)
    u = jnp.maximum(jnp.float32(TINY),
                    f * jnp.float32(1.0 - TINY) + jnp.float32(TINY))
    g = -jnp.log(-jnp.log(u))

    val = pred_ref[...] + g
    rowmax = jnp.max(val, axis=1, keepdims=True)
    cand = jnp.where(val == rowmax, col, jnp.int32(D))
    x1 = jnp.min(cand, axis=1)

    rctr = (jax.lax.broadcasted_iota(jnp.int32, (ROWS,), 0) + r0).astype(jnp.uint32)
    uj = jnp.maximum(jnp.float32(0.0), _bits_to_unit_float(_threefry_xor(K_JUMP, rctr)))

    t = t_ref[0]
    h = h_ref[0]
    inten = jnp.float32(1.0) / (jnp.float32(1.0) - t)
    p_jump = jnp.float32(1.0) - jnp.exp(jnp.full((ROWS,), (-h) * inten, jnp.float32))

    xt = xt_ref[...]
    mask = (uj < p_jump) & (x1 != xt)
    out_ref[...] = jnp.where(mask, x1, xt)


def kernel(pred_t, t, step_size, x_t):
    grid = (N // ROWS,)
    return pl.pallas_call(
        _body,
        grid=grid,
        in_specs=[
            pl.BlockSpec((ROWS, D), lambda i: (i, 0)),
            pl.BlockSpec((ROWS,), lambda i: (i,)),
            pl.BlockSpec(memory_space=pltpu.SMEM),
            pl.BlockSpec(memory_space=pltpu.SMEM),
        ],
        out_specs=pl.BlockSpec((ROWS,), lambda i: (i,)),
        out_shape=jax.ShapeDtypeStruct((N,), jnp.int32),
    )(pred_t, x_t, t, step_size)


# speedup vs baseline: 1.6050x; 1.0253x over previous
import numpy as np
import jax
import jax.numpy as jnp
from jax.experimental import pallas as pl
from jax.experimental.pallas import tpu as pltpu

N = 262144
D = 119
ROWS = 1024

K_SAMP = (1832780943, 270669613)
K_JUMP = (64467757, 2916123636)
TINY = float(np.finfo(np.float32).tiny)
MAGIC = 0x1BD11BDA

_ROT_A = (13, 15, 26, 6)
_ROT_B = (17, 29, 16, 24)


def _threefry_xor(key, ctr):
    k0, k1 = int(key[0]), int(key[1])
    ks = (k0, k1, k0 ^ k1 ^ MAGIC)
    x0 = jnp.full(ctr.shape, jnp.uint32(ks[0]), jnp.uint32)
    x1 = ctr + jnp.uint32(ks[1])
    for i in range(5):
        for r in (_ROT_A if i % 2 == 0 else _ROT_B):
            x0 = x0 + x1
            x1 = ((x1 << jnp.uint32(r)) | (x1 >> jnp.uint32(32 - r))) ^ x0
        x0 = x0 + jnp.uint32(ks[(i + 1) % 3])
        x1 = x1 + jnp.uint32((ks[(i + 2) % 3] + i + 1) & 0xFFFFFFFF)
    return x0 ^ x1


def _bits_to_unit_float(bits):
    fb = (bits >> jnp.uint32(9)) | jnp.uint32(0x3F800000)
    return jax.lax.bitcast_convert_type(fb, jnp.float32) - jnp.float32(1.0)


def _body(pred_ref, xt_ref, t_ref, h_ref, out_ref):
    pid = pl.program_id(0)
    r0 = pid * ROWS

    row = jax.lax.broadcasted_iota(jnp.int32, (ROWS, D), 0) + r0
    col = jax.lax.broadcasted_iota(jnp.int32, (ROWS, D), 1)
    ctr = (row * D + col).astype(jnp.uint32)
    bits = _threefry_xor(K_SAMP, ctr)
    f = _bits_to_unit_float(bits)
    u = jnp.maximum(jnp.float32(TINY),
                    f * jnp.float32(1.0 - TINY) + jnp.float32(TINY))
    g = -jnp.log(-jnp.log(u))

    val = pred_ref[...] + g
    rowmax = jnp.max(val, axis=1, keepdims=True)
    cand = jnp.where(val == rowmax, col, jnp.int32(D))
    x1 = jnp.min(cand, axis=1)

    rctr = (jax.lax.broadcasted_iota(jnp.int32, (ROWS,), 0) + r0).astype(jnp.uint32)
    uj = jnp.maximum(jnp.float32(0.0), _bits_to_unit_float(_threefry_xor(K_JUMP, rctr)))

    t = t_ref[0]
    h = h_ref[0]
    inten = jnp.float32(1.0) / (jnp.float32(1.0) - t)
    p_jump = jnp.float32(1.0) - jnp.exp(jnp.full((ROWS,), (-h) * inten, jnp.float32))

    xt = xt_ref[...]
    mask = (uj < p_jump) & (x1 != xt)
    out_ref[...] = jnp.where(mask, x1, xt)


def kernel(pred_t, t, step_size, x_t):
    grid = (N // ROWS,)
    return pl.pallas_call(
        _body,
        grid=grid,
        in_specs=[
            pl.BlockSpec((ROWS, D), lambda i: (i, 0)),
            pl.BlockSpec((ROWS,), lambda i: (i,)),
            pl.BlockSpec(memory_space=pltpu.SMEM),
            pl.BlockSpec(memory_space=pltpu.SMEM),
        ],
        out_specs=pl.BlockSpec((ROWS,), lambda i: (i,)),
        out_shape=jax.ShapeDtypeStruct((N,), jnp.int32),
        compiler_params=pltpu.CompilerParams(
            dimension_semantics=("parallel",)),
    )(pred_t, x_t, t, step_size)


# speedup vs baseline: 1.6498x; 1.0279x over previous
import numpy as np
import jax
import jax.numpy as jnp
from jax.experimental import pallas as pl
from jax.experimental.pallas import tpu as pltpu

N = 262144
D = 119
ROWS = 2048

K_SAMP = (1832780943, 270669613)
K_JUMP = (64467757, 2916123636)
TINY = float(np.finfo(np.float32).tiny)
MAGIC = 0x1BD11BDA

_ROT_A = (13, 15, 26, 6)
_ROT_B = (17, 29, 16, 24)


def _threefry_xor(key, ctr):
    k0, k1 = int(key[0]), int(key[1])
    ks = (k0, k1, k0 ^ k1 ^ MAGIC)
    x0 = jnp.full(ctr.shape, jnp.uint32(ks[0]), jnp.uint32)
    x1 = ctr + jnp.uint32(ks[1])
    for i in range(5):
        for r in (_ROT_A if i % 2 == 0 else _ROT_B):
            x0 = x0 + x1
            x1 = ((x1 << jnp.uint32(r)) | (x1 >> jnp.uint32(32 - r))) ^ x0
        x0 = x0 + jnp.uint32(ks[(i + 1) % 3])
        x1 = x1 + jnp.uint32((ks[(i + 2) % 3] + i + 1) & 0xFFFFFFFF)
    return x0 ^ x1


def _bits_to_unit_float(bits):
    fb = (bits >> jnp.uint32(9)) | jnp.uint32(0x3F800000)
    return jax.lax.bitcast_convert_type(fb, jnp.float32) - jnp.float32(1.0)


def _body(pred_ref, xt_ref, t_ref, h_ref, out_ref):
    pid = pl.program_id(0)
    r0 = pid * ROWS

    row = jax.lax.broadcasted_iota(jnp.int32, (ROWS, D), 0) + r0
    col = jax.lax.broadcasted_iota(jnp.int32, (ROWS, D), 1)
    ctr = (row * D + col).astype(jnp.uint32)
    bits = _threefry_xor(K_SAMP, ctr)
    f = _bits_to_unit_float(bits)
    u = jnp.maximum(jnp.float32(TINY),
                    f * jnp.float32(1.0 - TINY) + jnp.float32(TINY))
    e = -jnp.log(u)

    val = pred_ref[...] - jnp.log(e)
    rowmax = jnp.max(val, axis=1, keepdims=True)
    cand = jnp.where(val == rowmax, col, jnp.int32(D))
    x1 = jnp.min(cand, axis=1)

    rctr = (jax.lax.broadcasted_iota(jnp.int32, (ROWS,), 0) + r0).astype(jnp.uint32)
    uj = jnp.maximum(jnp.float32(0.0), _bits_to_unit_float(_threefry_xor(K_JUMP, rctr)))

    t = t_ref[0]
    h = h_ref[0]
    inten = jnp.float32(1.0) / (jnp.float32(1.0) - t)
    p_jump = jnp.float32(1.0) - jnp.exp(jnp.full((ROWS,), (-h) * inten, jnp.float32))

    xt = xt_ref[...]
    mask = (uj < p_jump) & (x1 != xt)
    out_ref[...] = jnp.where(mask, x1, xt)


def kernel(pred_t, t, step_size, x_t):
    grid = (N // ROWS,)
    return pl.pallas_call(
        _body,
        grid=grid,
        in_specs=[
            pl.BlockSpec((ROWS, D), lambda i: (i, 0)),
            pl.BlockSpec((ROWS,), lambda i: (i,)),
            pl.BlockSpec(memory_space=pltpu.SMEM),
            pl.BlockSpec(memory_space=pltpu.SMEM),
        ],
        out_specs=pl.BlockSpec((ROWS,), lambda i: (i,)),
        out_shape=jax.ShapeDtypeStruct((N,), jnp.int32),
        compiler_params=pltpu.CompilerParams(
            dimension_semantics=("parallel",)),
    )(pred_t, x_t, t, step_size)
